# baseline (speedup 1.0000x reference)
import functools

import jax
import jax.numpy as jnp
from jax import lax
from jax.experimental import pallas as pl
from jax.experimental.pallas import tpu as pltpu
from jax.experimental.pallas import tpu_sc as plsc

_N = 10000
_D = 256
_P = 10
_E = 160000
_NC = 2
_NS = 16
_NW = _NC * _NS
_WQ = 4
_Q = _D // (_NW * _WQ)
_SL = _N * _WQ
_CB = 2000
_NCB = _E // _CB
_NG = _CB // 4

_mesh = plsc.VectorSubcoreMesh(core_axis_name="c", subcore_axis_name="s")


def _rsqrt4(d):
    i = plsc.bitcast(d, jnp.int32)
    i = 0x5F3759DF - lax.shift_right_logical(i, 1)
    y = plsc.bitcast(i, jnp.float32)
    for _ in range(4):
        y = y * (1.5 - 0.5 * d * y * y)
    return y


@functools.partial(
    pl.kernel,
    mesh=_mesh,
    compiler_params=pltpu.CompilerParams(needs_layout_passes=False),
    out_type=(
        jax.ShapeDtypeStruct((_P + 1, _Q, _NW, _SL), jnp.float32),
        jax.ShapeDtypeStruct((_N,), jnp.float32),
    ),
    scratch_types=[
        pltpu.VMEM((_SL,), jnp.float32),
        pltpu.VMEM((_SL,), jnp.float32),
        pltpu.VMEM((_N,), jnp.float32),
        pltpu.VMEM((_N,), jnp.float32),
        pltpu.VMEM((_CB,), jnp.int32),
        pltpu.VMEM((16 * (_P + 1),), jnp.float32),
    ],
)
def _sc_propagate(x_hbm, pack_hbm, scales_hbm, out_hbm, nl_hbm,
                  h, agg, nl, nlr, chunk, scales):
    cid = lax.axis_index("c")
    sid = lax.axis_index("s")
    wid = sid * _NC + cid

    iota = lax.iota(jnp.int32, 16)
    pat2 = lax.shift_right_logical(iota, 2)
    iota4 = jnp.bitwise_and(iota, 3)
    zeros16 = jnp.zeros((16,), jnp.float32)
    ones16 = zeros16 + 1.0

    pltpu.sync_copy(scales_hbm, scales)

    def _zero_deg(i, _):
        agg[pl.ds(i * 16, 16)] = zeros16
        return 0
    lax.fori_loop(0, 2 * _N // 16, _zero_deg, 0)

    def _deg_blk(b, _):
        pltpu.sync_copy(pack_hbm.at[pl.ds(b * _CB, _CB)], chunk)

        def _grp(g, _):
            pk = plsc.load_gather(chunk, [g * 4 + pat2])
            src = lax.shift_right_logical(pk, 16)
            dst = jnp.bitwise_and(pk, 0xFFFF)
            plsc.addupdate_scatter(agg, [src], ones16)
            plsc.addupdate_scatter(agg, [dst + _N], ones16)
            return 0
        lax.fori_loop(0, _NG, _grp, 0)
        return 0
    lax.fori_loop(0, _NCB, _deg_blk, 0)

    def _norms(i, _):
        d_o = agg[pl.ds(i * 16, 16)]
        nl[pl.ds(i * 16, 16)] = 2.0 * _rsqrt4(jnp.maximum(d_o, 4.0))
        d_i = agg[pl.ds(_N + i * 16, 16)]
        nr_v = 2.0 * _rsqrt4(jnp.maximum(d_i, 4.0))
        nlr[pl.ds(i * 16, 16)] = nr_v
        return 0
    lax.fori_loop(0, _N // 16, _norms, 0)

    @pl.when(wid == 0)
    def _():
        pltpu.sync_copy(nl, nl_hbm)

    def _mul_nl(i, _):
        nlr[pl.ds(i * 16, 16)] = nlr[pl.ds(i * 16, 16)] * nl[pl.ds(i * 16, 16)]
        return 0
    lax.fori_loop(0, _N // 16, _mul_nl, 0)

    for q in range(_Q):
        pltpu.sync_copy(x_hbm.at[q, wid], h)
        c0 = scales[pl.ds(16 * _P, 16)]

        def _w0(i, _):
            nv = plsc.load_gather(nl, [i * 4 + pat2])
            h[pl.ds(i * 16, 16)] = h[pl.ds(i * 16, 16)] * nv * c0
            return 0
        lax.fori_loop(0, _SL // 16, _w0, 0)
        pltpu.sync_copy(h, out_hbm.at[0, q, wid])

        for p in range(_P):
            def _zero(i, _):
                agg[pl.ds(i * 16, 16)] = zeros16
                return 0
            lax.fori_loop(0, _SL // 16, _zero, 0)

            def _blk(b, _):
                pltpu.sync_copy(pack_hbm.at[pl.ds(b * _CB, _CB)], chunk)

                def _grp(g, _):
                    pk = plsc.load_gather(chunk, [g * 4 + pat2])
                    src = lax.shift_right_logical(pk, 16)
                    v = plsc.load_gather(h, [src * 4 + iota4])
                    dst = jnp.bitwise_and(pk, 0xFFFF)
                    plsc.addupdate_scatter(agg, [dst * 4 + iota4], v)
                    return 0
                lax.fori_loop(0, _NG, _grp, 0)
                return 0
            lax.fori_loop(0, _NCB, _blk, 0)

            rv = scales[pl.ds(16 * p, 16)]

            def _post(i, _):
                nv = plsc.load_gather(nlr, [i * 4 + pat2])
                h[pl.ds(i * 16, 16)] = agg[pl.ds(i * 16, 16)] * nv * rv
                return 0
            lax.fori_loop(0, _SL // 16, _post, 0)
            pltpu.sync_copy(h, out_hbm.at[p + 1, q, wid])


def _finish_body(w_ref, inv_ref, hid_ref, hl_ref):
    w = w_ref[...]
    s = inv_ref[...]
    scaled = w * s[None]
    hl_ref[...] = scaled
    hid_ref[...] = jnp.sum(scaled, axis=0)


def _finish(w_hl, invl):
    bn = 400
    return pl.pallas_call(
        _finish_body,
        grid=(_N // bn,),
        in_specs=[
            pl.BlockSpec((_P + 1, bn, _D), lambda i: (0, i, 0)),
            pl.BlockSpec((bn, 1), lambda i: (i, 0)),
        ],
        out_specs=[
            pl.BlockSpec((bn, _D), lambda i: (i, 0)),
            pl.BlockSpec((_P + 1, bn, _D), lambda i: (0, i, 0)),
        ],
        out_shape=[
            jax.ShapeDtypeStruct((_N, _D), jnp.float32),
            jax.ShapeDtypeStruct((_P + 1, _N, _D), jnp.float32),
        ],
    )(w_hl, invl)


def kernel(x, edge_index, coes):
    src = edge_index[0].astype(jnp.int32)
    dst = edge_index[1].astype(jnp.int32)
    pack = jnp.left_shift(src, 16) | dst
    x_sc = x.reshape(_N, _Q, _NW, _WQ).transpose(1, 2, 0, 3).reshape(_Q, _NW, _SL)
    ratios = coes[1:] / coes[:-1]
    scales = jnp.concatenate([ratios, coes[:1]])
    scales16 = jnp.broadcast_to(scales[:, None], (_P + 1, 16)).reshape(-1)
    out_sc, nl = _sc_propagate(x_sc, pack, scales16)
    w_hl = (out_sc.reshape(_P + 1, _Q, _NW, _N, _WQ)
            .transpose(0, 3, 1, 2, 4).reshape(_P + 1, _N, _D))
    invl = (1.0 / nl).reshape(_N, 1)
    hidden, hl = _finish(w_hl, invl)
    return (hidden, hl)


# speedup vs baseline: 2.6512x; 2.6512x over previous
import functools

import jax
import jax.numpy as jnp
from jax import lax
from jax.experimental import pallas as pl
from jax.experimental.pallas import tpu as pltpu
from jax.experimental.pallas import tpu_sc as plsc

_N = 10000
_D = 256
_P = 10
_E = 160000
_NC = 2
_NS = 16
_NW = _NC * _NS
_WQ = 4
_Q = _D // (_NW * _WQ)
_SL = _N * _WQ
_CB = 2000
_NCB = _E // _CB
_NG = _CB // 4

_mesh = plsc.VectorSubcoreMesh(core_axis_name="c", subcore_axis_name="s")


def _rsqrt4(d):
    i = plsc.bitcast(d, jnp.int32)
    i = 0x5F3759DF - lax.shift_right_logical(i, 1)
    y = plsc.bitcast(i, jnp.float32)
    for _ in range(4):
        y = y * (1.5 - 0.5 * d * y * y)
    return y


@functools.partial(
    pl.kernel,
    mesh=_mesh,
    compiler_params=pltpu.CompilerParams(needs_layout_passes=False),
    out_type=(
        jax.ShapeDtypeStruct((_P + 1, _Q, _NW, _SL), jnp.float32),
        jax.ShapeDtypeStruct((_N,), jnp.float32),
    ),
    scratch_types=[
        pltpu.VMEM((_SL,), jnp.float32),
        pltpu.VMEM((_SL,), jnp.float32),
        pltpu.VMEM((_N,), jnp.float32),
        pltpu.VMEM((_N,), jnp.float32),
        pltpu.VMEM((_CB,), jnp.int32),
        pltpu.VMEM((16 * (_P + 1),), jnp.float32),
    ],
)
def _sc_propagate(x_hbm, pack_hbm, scales_hbm, out_hbm, nl_hbm,
                  h, agg, nl, nlr, chunk, scales):
    cid = lax.axis_index("c")
    sid = lax.axis_index("s")
    wid = sid * _NC + cid

    iota = lax.iota(jnp.int32, 16)
    pat2 = lax.shift_right_logical(iota, 2)
    iota4 = jnp.bitwise_and(iota, 3)
    zeros16 = jnp.zeros((16,), jnp.float32)
    ones16 = zeros16 + 1.0

    pltpu.sync_copy(scales_hbm, scales)

    def _zero_deg(i, _):
        agg[pl.ds(i * 16, 16)] = zeros16
        return 0
    lax.fori_loop(0, 2 * _N // 16, _zero_deg, 0)

    def _deg_blk(b, _):
        pltpu.sync_copy(pack_hbm.at[pl.ds(b * _CB, _CB)], chunk)

        @plsc.parallel_loop(0, _NG, unroll=8)
        def _grp(g):
            pk = plsc.load_gather(chunk, [g * 4 + pat2])
            src = lax.shift_right_logical(pk, 16)
            dst = jnp.bitwise_and(pk, 0xFFFF)
            plsc.addupdate_scatter(agg, [src], ones16)
            plsc.addupdate_scatter(agg, [dst + _N], ones16)
        return 0
    lax.fori_loop(0, _NCB, _deg_blk, 0)

    def _norms(i, _):
        d_o = agg[pl.ds(i * 16, 16)]
        nl[pl.ds(i * 16, 16)] = 2.0 * _rsqrt4(jnp.maximum(d_o, 4.0))
        d_i = agg[pl.ds(_N + i * 16, 16)]
        nr_v = 2.0 * _rsqrt4(jnp.maximum(d_i, 4.0))
        nlr[pl.ds(i * 16, 16)] = nr_v
        return 0
    lax.fori_loop(0, _N // 16, _norms, 0)

    @pl.when(wid == 0)
    def _():
        pltpu.sync_copy(nl, nl_hbm)

    def _mul_nl(i, _):
        nlr[pl.ds(i * 16, 16)] = nlr[pl.ds(i * 16, 16)] * nl[pl.ds(i * 16, 16)]
        return 0
    lax.fori_loop(0, _N // 16, _mul_nl, 0)

    for q in range(_Q):
        pltpu.sync_copy(x_hbm.at[q, wid], h)
        c0 = scales[pl.ds(16 * _P, 16)]

        def _w0(i, _):
            nv = plsc.load_gather(nl, [i * 4 + pat2])
            h[pl.ds(i * 16, 16)] = h[pl.ds(i * 16, 16)] * nv * c0
            return 0
        lax.fori_loop(0, _SL // 16, _w0, 0)
        pltpu.sync_copy(h, out_hbm.at[0, q, wid])

        for p in range(_P):
            def _zero(i, _):
                agg[pl.ds(i * 16, 16)] = zeros16
                return 0
            lax.fori_loop(0, _SL // 16, _zero, 0)

            def _blk(b, _):
                pltpu.sync_copy(pack_hbm.at[pl.ds(b * _CB, _CB)], chunk)

                @plsc.parallel_loop(0, _NG, unroll=8)
                def _grp(g):
                    pk = plsc.load_gather(chunk, [g * 4 + pat2])
                    src = lax.shift_right_logical(pk, 16)
                    v = plsc.load_gather(h, [src * 4 + iota4])
                    dst = jnp.bitwise_and(pk, 0xFFFF)
                    plsc.addupdate_scatter(agg, [dst * 4 + iota4], v)
                return 0
            lax.fori_loop(0, _NCB, _blk, 0)

            rv = scales[pl.ds(16 * p, 16)]

            def _post(i, _):
                nv = plsc.load_gather(nlr, [i * 4 + pat2])
                h[pl.ds(i * 16, 16)] = agg[pl.ds(i * 16, 16)] * nv * rv
                return 0
            lax.fori_loop(0, _SL // 16, _post, 0)
            pltpu.sync_copy(h, out_hbm.at[p + 1, q, wid])


def _finish_body(w_ref, inv_ref, hid_ref, hl_ref):
    w = w_ref[...]
    s = inv_ref[...]
    scaled = w * s[None]
    hl_ref[...] = scaled
    hid_ref[...] = jnp.sum(scaled, axis=0)


def _finish(w_hl, invl):
    bn = 400
    return pl.pallas_call(
        _finish_body,
        grid=(_N // bn,),
        in_specs=[
            pl.BlockSpec((_P + 1, bn, _D), lambda i: (0, i, 0)),
            pl.BlockSpec((bn, 1), lambda i: (i, 0)),
        ],
        out_specs=[
            pl.BlockSpec((bn, _D), lambda i: (i, 0)),
            pl.BlockSpec((_P + 1, bn, _D), lambda i: (0, i, 0)),
        ],
        out_shape=[
            jax.ShapeDtypeStruct((_N, _D), jnp.float32),
            jax.ShapeDtypeStruct((_P + 1, _N, _D), jnp.float32),
        ],
    )(w_hl, invl)


def kernel(x, edge_index, coes):
    src = edge_index[0].astype(jnp.int32)
    dst = edge_index[1].astype(jnp.int32)
    pack = jnp.left_shift(src, 16) | dst
    x_sc = x.reshape(_N, _Q, _NW, _WQ).transpose(1, 2, 0, 3).reshape(_Q, _NW, _SL)
    ratios = coes[1:] / coes[:-1]
    scales = jnp.concatenate([ratios, coes[:1]])
    scales16 = jnp.broadcast_to(scales[:, None], (_P + 1, 16)).reshape(-1)
    out_sc, nl = _sc_propagate(x_sc, pack, scales16)
    w_hl = (out_sc.reshape(_P + 1, _Q, _NW, _N, _WQ)
            .transpose(0, 3, 1, 2, 4).reshape(_P + 1, _N, _D))
    invl = (1.0 / nl).reshape(_N, 1)
    hidden, hl = _finish(w_hl, invl)
    return (hidden, hl)
